# baseline (speedup 1.0000x reference)
import functools

import jax
import jax.numpy as jnp
from jax import lax
from jax.experimental import pallas as pl
from jax.experimental.pallas import tpu as pltpu
from jax.experimental.pallas import tpu_sc as plsc

ALG = 248
NC = 2
NS = 16
NW = NC * NS
BC = 128
LANES = 16
COO_CHUNK = 4800


def _sc_bracket(nch, ncoo):
    blk = BC * ALG
    mesh = plsc.VectorSubcoreMesh(core_axis_name="c", subcore_axis_name="s")

    @functools.partial(
        pl.kernel,
        out_type=jax.ShapeDtypeStruct((NW * nch, blk), jnp.float32),
        mesh=mesh,
        compiler_params=pltpu.CompilerParams(needs_layout_passes=False),
        scratch_types=[
            pltpu.VMEM((blk,), jnp.float32),
            pltpu.VMEM((blk,), jnp.float32),
            pltpu.VMEM((blk,), jnp.float32),
            pltpu.VMEM((COO_CHUNK,), jnp.int32),
            pltpu.VMEM((COO_CHUNK,), jnp.int32),
            pltpu.VMEM((COO_CHUNK,), jnp.int32),
            pltpu.VMEM((COO_CHUNK,), jnp.float32),
            pltpu.VMEM((LANES,), jnp.float32),
        ],
    )
    def kfn(x_hbm, y_hbm, al_hbm, ci_hbm, cj_hbm, ck_hbm, cv_hbm, out_hbm,
            xv, yv, ov, civ, cjv, ckv, cvv, alv):
        wid = lax.axis_index("c") * NS + lax.axis_index("s")
        pltpu.sync_copy(al_hbm, alv)

        def chunk_body(ch, _):
            row = wid * nch + ch
            pltpu.sync_copy(x_hbm.at[row], xv)
            pltpu.sync_copy(y_hbm.at[row], yv)

            def zero_body(z, _z):
                ov[pl.ds(z * LANES, LANES)] = jnp.zeros((LANES,), jnp.float32)
                return _z
            lax.fori_loop(0, blk // LANES, zero_body, 0)

            def coo_body(t, _t):
                pltpu.sync_copy(ci_hbm.at[t], civ)
                pltpu.sync_copy(cj_hbm.at[t], cjv)
                pltpu.sync_copy(ck_hbm.at[t], ckv)
                pltpu.sync_copy(cv_hbm.at[t], cvv)

                def group_body(grp, _g):
                    base = grp * LANES
                    ivg = civ[pl.ds(base, LANES)]
                    jvg = cjv[pl.ds(base, LANES)]
                    kvg = ckv[pl.ds(base, LANES)]
                    vbg = cvv[pl.ds(base, LANES)] * alv[...]
                    for l in range(LANES):
                        io = ivg[l]
                        jo = jvg[l]
                        ko = kvg[l]
                        vb = jnp.broadcast_to(vbg[l], (LANES,))
                        for g in range(BC // LANES):
                            o = g * LANES
                            xi = xv[pl.ds(io + o, LANES)]
                            yj = yv[pl.ds(jo + o, LANES)]
                            xj = xv[pl.ds(jo + o, LANES)]
                            yi = yv[pl.ds(io + o, LANES)]
                            plsc.addupdate(ov.at[pl.ds(ko + o, LANES)],
                                           vb * (xi * yj - xj * yi))
                    return _g
                lax.fori_loop(0, COO_CHUNK // LANES, group_body, 0)
                return _t
            lax.fori_loop(0, ncoo, coo_body, 0)

            pltpu.sync_copy(ov, out_hbm.at[row])
            return _
        lax.fori_loop(0, nch, chunk_body, 0)

    return kfn


def kernel(x, y, alpha, coo_i, coo_j, coo_k, coo_vals):
    B = x.shape[0]
    nch = B // (NW * BC)

    nh = coo_i.shape[0] // 2
    coo_i, coo_j = coo_i[:nh], coo_j[:nh]
    coo_k, coo_vals = coo_k[:nh], coo_vals[:nh]

    ncoo = -(-nh // COO_CHUNK)
    pad = ncoo * COO_CHUNK - nh
    if pad:
        zi = jnp.zeros((pad,), jnp.int32)
        coo_i = jnp.concatenate([coo_i, zi])
        coo_j = jnp.concatenate([coo_j, zi])
        coo_k = jnp.concatenate([coo_k, zi])
        coo_vals = jnp.concatenate([coo_vals, jnp.zeros((pad,), jnp.float32)])

    ci = (coo_i * BC).reshape(ncoo, COO_CHUNK)
    cj = (coo_j * BC).reshape(ncoo, COO_CHUNK)
    ck = (coo_k * BC).reshape(ncoo, COO_CHUNK)
    cv = coo_vals.reshape(ncoo, COO_CHUNK)
    al = jnp.full((LANES,), alpha, jnp.float32)

    xb = x.reshape(NW * nch, BC, ALG).transpose(0, 2, 1).reshape(
        NW * nch, ALG * BC)
    yb = y.reshape(NW * nch, BC, ALG).transpose(0, 2, 1).reshape(
        NW * nch, ALG * BC)

    outb = _sc_bracket(nch, ncoo)(xb, yb, al, ci, cj, ck, cv)
    return outb.reshape(NW * nch, ALG, BC).transpose(0, 2, 1).reshape(B, ALG)


# speedup vs baseline: 1.9649x; 1.9649x over previous
import functools

import jax
import jax.numpy as jnp
from jax import lax
from jax.experimental import pallas as pl
from jax.experimental.pallas import tpu as pltpu
from jax.experimental.pallas import tpu_sc as plsc

ALG = 248
NC = 2
NS = 16
NW = NC * NS
BC = 128
LANES = 16
COO_CHUNK = 4800
BU = 2


def _sc_bracket(nch, ncoo):
    blk = BC * ALG
    mesh = plsc.VectorSubcoreMesh(core_axis_name="c", subcore_axis_name="s")

    @functools.partial(
        pl.kernel,
        out_type=jax.ShapeDtypeStruct((NW * nch, blk), jnp.float32),
        mesh=mesh,
        compiler_params=pltpu.CompilerParams(needs_layout_passes=False),
        scratch_types=[
            pltpu.VMEM((blk,), jnp.float32),
            pltpu.VMEM((blk,), jnp.float32),
            pltpu.VMEM((blk,), jnp.float32),
            pltpu.VMEM((COO_CHUNK,), jnp.int32),
            pltpu.VMEM((COO_CHUNK,), jnp.int32),
            pltpu.VMEM((COO_CHUNK,), jnp.int32),
            pltpu.VMEM((COO_CHUNK,), jnp.float32),
            pltpu.VMEM((LANES,), jnp.float32),
        ],
    )
    def kfn(x_hbm, y_hbm, al_hbm, ci_hbm, cj_hbm, ck_hbm, cv_hbm, out_hbm,
            xv, yv, ov, civ, cjv, ckv, cvv, alv):
        wid = lax.axis_index("c") * NS + lax.axis_index("s")
        pltpu.sync_copy(al_hbm, alv)

        def chunk_body(ch, _):
            row = wid * nch + ch
            pltpu.sync_copy(x_hbm.at[row], xv)
            pltpu.sync_copy(y_hbm.at[row], yv)

            def zero_body(z, _z):
                ov[pl.ds(z * LANES, LANES)] = jnp.zeros((LANES,), jnp.float32)
                return _z
            lax.fori_loop(0, blk // LANES, zero_body, 0)

            def coo_body(t, _t):
                pltpu.sync_copy(ci_hbm.at[t], civ)
                pltpu.sync_copy(cj_hbm.at[t], cjv)
                pltpu.sync_copy(ck_hbm.at[t], ckv)
                pltpu.sync_copy(cv_hbm.at[t], cvv)

                def group_body(g, _g):
                    iv = civ[pl.ds(g * LANES, LANES)]
                    jv = cjv[pl.ds(g * LANES, LANES)]
                    kv = ckv[pl.ds(g * LANES, LANES)]
                    vv = cvv[pl.ds(g * LANES, LANES)] * alv[...]

                    @plsc.parallel_loop(0, BC, 1, unroll=BU)
                    def batch_body(b):
                        off = b * ALG
                        xr = xv.at[pl.ds(off, ALG)]
                        yr = yv.at[pl.ds(off, ALG)]
                        xi = plsc.load_gather(xr, [iv])
                        yj = plsc.load_gather(yr, [jv])
                        xj = plsc.load_gather(xr, [jv])
                        yi = plsc.load_gather(yr, [iv])
                        plsc.addupdate_scatter(
                            ov.at[pl.ds(off, ALG)], [kv],
                            vv * (xi * yj - xj * yi))
                    return _g
                lax.fori_loop(0, COO_CHUNK // LANES, group_body, 0)
                return _t
            lax.fori_loop(0, ncoo, coo_body, 0)

            pltpu.sync_copy(ov, out_hbm.at[row])
            return _
        lax.fori_loop(0, nch, chunk_body, 0)

    return kfn


def kernel(x, y, alpha, coo_i, coo_j, coo_k, coo_vals):
    B = x.shape[0]
    nch = B // (NW * BC)

    nh = coo_i.shape[0] // 2
    coo_i, coo_j = coo_i[:nh], coo_j[:nh]
    coo_k, coo_vals = coo_k[:nh], coo_vals[:nh]

    ncoo = -(-nh // COO_CHUNK)
    pad = ncoo * COO_CHUNK - nh
    if pad:
        zi = jnp.zeros((pad,), jnp.int32)
        coo_i = jnp.concatenate([coo_i, zi])
        coo_j = jnp.concatenate([coo_j, zi])
        coo_k = jnp.concatenate([coo_k, zi])
        coo_vals = jnp.concatenate([coo_vals, jnp.zeros((pad,), jnp.float32)])

    xb = x.reshape(NW * nch, BC * ALG)
    yb = y.reshape(NW * nch, BC * ALG)
    ci = coo_i.reshape(ncoo, COO_CHUNK)
    cj = coo_j.reshape(ncoo, COO_CHUNK)
    ck = coo_k.reshape(ncoo, COO_CHUNK)
    cv = coo_vals.reshape(ncoo, COO_CHUNK)
    al = jnp.full((LANES,), alpha, jnp.float32)

    outb = _sc_bracket(nch, ncoo)(xb, yb, al, ci, cj, ck, cv)
    return outb.reshape(B, ALG)


# speedup vs baseline: 2.0798x; 1.0585x over previous
import functools

import jax
import jax.numpy as jnp
from jax import lax
from jax.experimental import pallas as pl
from jax.experimental.pallas import tpu as pltpu
from jax.experimental.pallas import tpu_sc as plsc

ALG = 248
NC = 2
NS = 16
NW = NC * NS
BC = 128
LANES = 16
COO_CHUNK = 4800
BU = 2


def _sc_bracket(nch, ncoo):
    blk = BC * ALG
    mesh = plsc.VectorSubcoreMesh(core_axis_name="c", subcore_axis_name="s")

    @functools.partial(
        pl.kernel,
        out_type=jax.ShapeDtypeStruct((NW * nch, blk), jnp.float32),
        mesh=mesh,
        compiler_params=pltpu.CompilerParams(needs_layout_passes=False),
        scratch_types=[
            pltpu.VMEM((blk,), jnp.float32),
            pltpu.VMEM((blk,), jnp.float32),
            pltpu.VMEM((blk,), jnp.float32),
            pltpu.VMEM((COO_CHUNK,), jnp.int32),
            pltpu.VMEM((COO_CHUNK,), jnp.int32),
            pltpu.VMEM((COO_CHUNK,), jnp.int32),
            pltpu.VMEM((COO_CHUNK,), jnp.float32),
            pltpu.VMEM((LANES,), jnp.float32),
        ],
    )
    def kfn(x_hbm, y_hbm, al_hbm, ci_hbm, cj_hbm, ck_hbm, cv_hbm, out_hbm,
            xv, yv, ov, civ, cjv, ckv, cvv, alv):
        wid = lax.axis_index("c") * NS + lax.axis_index("s")
        pltpu.sync_copy(al_hbm, alv)

        def chunk_body(ch, _):
            row = wid * nch + ch
            pltpu.sync_copy(x_hbm.at[row], xv)
            pltpu.sync_copy(y_hbm.at[row], yv)

            def zero_body(z, _z):
                ov[pl.ds(z * LANES, LANES)] = jnp.zeros((LANES,), jnp.float32)
                return _z
            lax.fori_loop(0, blk // LANES, zero_body, 0)

            def coo_body(t, _t):
                pltpu.sync_copy(ci_hbm.at[t], civ)
                pltpu.sync_copy(cj_hbm.at[t], cjv)
                pltpu.sync_copy(ck_hbm.at[t], ckv)
                pltpu.sync_copy(cv_hbm.at[t], cvv)

                def group_body(g, _g):
                    iv = civ[pl.ds(g * LANES, LANES)]
                    jv = cjv[pl.ds(g * LANES, LANES)]
                    kv = ckv[pl.ds(g * LANES, LANES)]
                    vv = cvv[pl.ds(g * LANES, LANES)] * alv[...]

                    @plsc.parallel_loop(0, BC, 1, unroll=BU)
                    def batch_body(b):
                        bb = jnp.broadcast_to(b * ALG, (LANES,)).astype(jnp.int32)
                        gi = bb + iv
                        gj = bb + jv
                        xi = plsc.load_gather(xv, [gi])
                        yj = plsc.load_gather(yv, [gj])
                        xj = plsc.load_gather(xv, [gj])
                        yi = plsc.load_gather(yv, [gi])
                        plsc.addupdate_scatter(
                            ov, [bb + kv], vv * (xi * yj - xj * yi))
                    return _g
                lax.fori_loop(0, COO_CHUNK // LANES, group_body, 0)
                return _t
            lax.fori_loop(0, ncoo, coo_body, 0)

            pltpu.sync_copy(ov, out_hbm.at[row])
            return _
        lax.fori_loop(0, nch, chunk_body, 0)

    return kfn


def kernel(x, y, alpha, coo_i, coo_j, coo_k, coo_vals):
    B = x.shape[0]
    nch = B // (NW * BC)

    nh = coo_i.shape[0] // 2
    coo_i, coo_j = coo_i[:nh], coo_j[:nh]
    coo_k, coo_vals = coo_k[:nh], coo_vals[:nh]

    nbk = 8
    bi = coo_i % nbk
    bj = (coo_j + bi) % nbk
    order = jnp.argsort(bi * (2 * nbk) + bj, stable=True)
    counts = jnp.bincount(bi, length=nbk)
    offs = jnp.concatenate([jnp.zeros((1,), counts.dtype),
                            jnp.cumsum(counts)[:-1]])
    rank = jnp.arange(nh) - offs[bi[order]]
    perm = order[jnp.argsort(rank * nbk + bi[order], stable=True)]
    coo_i, coo_j = coo_i[perm], coo_j[perm]
    coo_k, coo_vals = coo_k[perm], coo_vals[perm]

    ncoo = -(-nh // COO_CHUNK)
    pad = ncoo * COO_CHUNK - nh
    if pad:
        zi = jnp.zeros((pad,), jnp.int32)
        coo_i = jnp.concatenate([coo_i, zi])
        coo_j = jnp.concatenate([coo_j, zi])
        coo_k = jnp.concatenate([coo_k, zi])
        coo_vals = jnp.concatenate([coo_vals, jnp.zeros((pad,), jnp.float32)])

    xb = x.reshape(NW * nch, BC * ALG)
    yb = y.reshape(NW * nch, BC * ALG)
    ci = coo_i.reshape(ncoo, COO_CHUNK)
    cj = coo_j.reshape(ncoo, COO_CHUNK)
    ck = coo_k.reshape(ncoo, COO_CHUNK)
    cv = coo_vals.reshape(ncoo, COO_CHUNK)
    al = jnp.full((LANES,), alpha, jnp.float32)

    outb = _sc_bracket(nch, ncoo)(xb, yb, al, ci, cj, ck, cv)
    return outb.reshape(B, ALG)


# speedup vs baseline: 2.0814x; 1.0008x over previous
import functools

import jax
import jax.numpy as jnp
from jax import lax
from jax.experimental import pallas as pl
from jax.experimental.pallas import tpu as pltpu
from jax.experimental.pallas import tpu_sc as plsc

ALG = 248
NC = 2
NS = 16
NW = NC * NS
BC = 128
LANES = 16
COO_CHUNK = 4800
BU = 2


def _sc_bracket(nch, ncoo):
    blk = BC * ALG
    mesh = plsc.VectorSubcoreMesh(core_axis_name="c", subcore_axis_name="s")

    @functools.partial(
        pl.kernel,
        out_type=jax.ShapeDtypeStruct((NW * nch, blk), jnp.float32),
        mesh=mesh,
        compiler_params=pltpu.CompilerParams(needs_layout_passes=False),
        scratch_types=[
            pltpu.VMEM((blk,), jnp.float32),
            pltpu.VMEM((blk,), jnp.float32),
            pltpu.VMEM((blk,), jnp.float32),
            pltpu.VMEM((COO_CHUNK,), jnp.int32),
            pltpu.VMEM((COO_CHUNK,), jnp.int32),
            pltpu.VMEM((COO_CHUNK,), jnp.int32),
            pltpu.VMEM((COO_CHUNK,), jnp.float32),
            pltpu.VMEM((LANES,), jnp.float32),
        ],
    )
    def kfn(x_hbm, y_hbm, al_hbm, ci_hbm, cj_hbm, ck_hbm, cv_hbm, out_hbm,
            xv, yv, ov, civ, cjv, ckv, cvv, alv):
        wid = lax.axis_index("c") * NS + lax.axis_index("s")
        pltpu.sync_copy(al_hbm, alv)

        def chunk_body(ch, _):
            row = wid * nch + ch
            pltpu.sync_copy(x_hbm.at[row], xv)
            pltpu.sync_copy(y_hbm.at[row], yv)

            def zero_body(z, _z):
                ov[pl.ds(z * LANES, LANES)] = jnp.zeros((LANES,), jnp.float32)
                return _z
            lax.fori_loop(0, blk // LANES, zero_body, 0)

            def coo_body(t, _t):
                pltpu.sync_copy(ci_hbm.at[t], civ)
                pltpu.sync_copy(cj_hbm.at[t], cjv)
                pltpu.sync_copy(ck_hbm.at[t], ckv)
                pltpu.sync_copy(cv_hbm.at[t], cvv)

                def group_body(g, _g):
                    iv = civ[pl.ds(g * LANES, LANES)]
                    jv = cjv[pl.ds(g * LANES, LANES)]
                    kv = ckv[pl.ds(g * LANES, LANES)]
                    vv = cvv[pl.ds(g * LANES, LANES)] * alv[...]

                    @plsc.parallel_loop(0, BC, 1, unroll=BU)
                    def batch_body(b):
                        bb = jnp.broadcast_to(b * ALG, (LANES,)).astype(jnp.int32)
                        gi = bb + iv
                        gj = bb + jv
                        xi = plsc.load_gather(xv, [gi])
                        yj = plsc.load_gather(yv, [gj])
                        xj = plsc.load_gather(xv, [gj])
                        yi = plsc.load_gather(yv, [gi])
                        plsc.addupdate_scatter(
                            ov, [bb + kv], vv * (xi * yj - xj * yi))
                    return _g
                lax.fori_loop(0, COO_CHUNK // LANES, group_body, 0)
                return _t
            lax.fori_loop(0, ncoo, coo_body, 0)

            pltpu.sync_copy(ov, out_hbm.at[row])
            return _
        lax.fori_loop(0, nch, chunk_body, 0)

    return kfn


def kernel(x, y, alpha, coo_i, coo_j, coo_k, coo_vals):
    B = x.shape[0]
    nch = B // (NW * BC)

    nh = coo_i.shape[0] // 2
    coo_i, coo_j = coo_i[:nh], coo_j[:nh]
    coo_k, coo_vals = coo_k[:nh], coo_vals[:nh]

    nbk = 8
    bi = coo_i % nbk
    bj = (coo_j + bi) % nbk
    bk = (coo_k + bi) % nbk
    order = jnp.argsort((bi * nbk + bj) * nbk + bk, stable=True)
    counts = jnp.bincount(bi, length=nbk)
    offs = jnp.concatenate([jnp.zeros((1,), counts.dtype),
                            jnp.cumsum(counts)[:-1]])
    rank = jnp.arange(nh) - offs[bi[order]]
    perm = order[jnp.argsort(rank * nbk + bi[order], stable=True)]
    coo_i, coo_j = coo_i[perm], coo_j[perm]
    coo_k, coo_vals = coo_k[perm], coo_vals[perm]

    ncoo = -(-nh // COO_CHUNK)
    pad = ncoo * COO_CHUNK - nh
    if pad:
        zi = jnp.zeros((pad,), jnp.int32)
        coo_i = jnp.concatenate([coo_i, zi])
        coo_j = jnp.concatenate([coo_j, zi])
        coo_k = jnp.concatenate([coo_k, zi])
        coo_vals = jnp.concatenate([coo_vals, jnp.zeros((pad,), jnp.float32)])

    xb = x.reshape(NW * nch, BC * ALG)
    yb = y.reshape(NW * nch, BC * ALG)
    ci = coo_i.reshape(ncoo, COO_CHUNK)
    cj = coo_j.reshape(ncoo, COO_CHUNK)
    ck = coo_k.reshape(ncoo, COO_CHUNK)
    cv = coo_vals.reshape(ncoo, COO_CHUNK)
    al = jnp.full((LANES,), alpha, jnp.float32)

    outb = _sc_bracket(nch, ncoo)(xb, yb, al, ci, cj, ck, cv)
    return outb.reshape(B, ALG)
